# Initial kernel scaffold; baseline (speedup 1.0000x reference)
import jax
import jax.numpy as jnp
from jax.experimental import pallas as pl


def kernel(similarity_map, ref_mask, original_size):
    raise NotImplementedError("write your pallas kernel here")


# speedup vs baseline: 24.6771x; 24.6771x over previous
import dataclasses

import jax
import jax.numpy as jnp
from jax.experimental import pallas as pl
from jax.experimental.pallas import tpu as pltpu
from jax.experimental.pallas import tpu_sc as plsc

_SC_PARAMS = pltpu.CompilerParams()
if "needs_layout_passes" in pltpu.CompilerParams.__dataclass_fields__:
    _SC_PARAMS = dataclasses.replace(_SC_PARAMS, needs_layout_passes=False)

N = 1024
FEAT = 32
NUM_FG = 40
NUM_BG = 2
NPTS = 48
L = 16
NCH = N // L
NEG_INF = float("-inf")
BIG_I = 1 << 30


def _prep_body(sim_ref, refneg_ref, rowmax_ref, rowarg_ref, colmax_ref,
               colarg_ref, colsum_ref, inv_ref):
    sim = sim_ref[...]
    refneg = refneg_ref[...]
    masked = sim + refneg
    rmax = jnp.max(masked, axis=1, keepdims=True)
    lane_iota = jax.lax.broadcasted_iota(jnp.int32, (N, N), 1)
    rowmax_ref[...] = rmax
    rowarg_ref[...] = jnp.min(
        jnp.where(masked == rmax, lane_iota, N), axis=1, keepdims=True)
    cmax = jnp.max(sim, axis=0, keepdims=True)
    row_iota = jax.lax.broadcasted_iota(jnp.int32, (N, N), 0)
    colmax_ref[...] = cmax
    colarg_ref[...] = jnp.min(
        jnp.where(sim == cmax, row_iota, N), axis=0, keepdims=True)
    mask01 = (refneg == 0.0).astype(jnp.float32)
    colsum_ref[...] = jnp.sum(sim * mask01, axis=0, keepdims=True)
    inv_ref[...] = jnp.full((1, 128), 1.0 / jnp.sum(mask01), jnp.float32)


def _lanes():
    return jax.lax.iota(jnp.int32, L)


def _splat(x):
    return jax.lax.broadcast(x, (L,))


def _chunk_of(i):
    return jax.lax.shift_right_logical(i, 4)


def _get_f32(ref, i):
    return jnp.max(plsc.load_gather(ref, [_splat(i)]))


def _get_i32(ref, i):
    return jnp.max(plsc.load_gather(ref, [_splat(i)]))


def _set_elem(ref, i, val):
    plsc.store_scatter(ref, [_splat(i)], _splat(val), mask=_lanes() == 0)


def _argmax_ref(ref):
    lanes = _lanes()

    def body(q, carry):
        vmv, vmi = carry
        v = ref[pl.ds(q * L, L)]
        m = v > vmv
        return jnp.where(m, v, vmv), jnp.where(m, lanes + q * L, vmi)

    vmv, vmi = jax.lax.fori_loop(1, NCH, body, (ref[pl.ds(0, L)], lanes))
    gmax = jnp.max(vmv)
    idx = jnp.min(jnp.where(vmv == gmax, vmi, jnp.int32(BIG_I)))
    return gmax, idx


def _argmax_hier(rb, chmax):
    lanes = _lanes()
    vmv = chmax[pl.ds(0, L)]
    vmi = lanes
    for g in range(1, NCH // L):
        v = chmax[pl.ds(g * L, L)]
        m = v > vmv
        vmv = jnp.where(m, v, vmv)
        vmi = jnp.where(m, lanes + g * L, vmi)
    gmax = jnp.max(vmv)
    q = jnp.min(jnp.where(vmv == gmax, vmi, jnp.int32(BIG_I)))
    v = plsc.load_gather(rb, [q * L + lanes])
    r = q * L + jnp.min(jnp.where(v == gmax, lanes, jnp.int32(BIG_I)))
    return gmax, r


def _update_chmax(rb, chmax, q):
    v = plsc.load_gather(rb, [q * L + _lanes()])
    _set_elem(chmax, q, jnp.max(v))


def _fill(ref, val):
    def body(q, _):
        ref[pl.ds(q * L, L)] = jnp.full((L,), val, ref.dtype)
        return 0
    jax.lax.fori_loop(0, ref.shape[0] // L, body, 0)


def _refresh(src_hbm, r, rb, ra, cneg, chmax, rowbuf, sem):
    pltpu.async_copy(src_hbm.at[r], rowbuf, sem).wait()
    lanes = _lanes()

    def body(q, carry):
        vmv, vmi = carry
        v = rowbuf[pl.ds(q * L, L)] + cneg[pl.ds(q * L, L)]
        m = v > vmv
        return jnp.where(m, v, vmv), jnp.where(m, lanes + q * L, vmi)

    init = (rowbuf[pl.ds(0, L)] + cneg[pl.ds(0, L)], lanes)
    vmv, vmi = jax.lax.fori_loop(1, NCH, body, init)
    gmax = jnp.max(vmv)
    idx = jnp.min(jnp.where(vmv == gmax, vmi, jnp.int32(BIG_I)))
    _set_elem(rb, r, gmax)
    _set_elem(ra, r, idx)
    _update_chmax(rb, chmax, _chunk_of(r))


def _greedy(src_hbm, num_valid, rb, ra, cneg, chmax, rowbuf, sem, record):
    def pop():
        gmax, r = _argmax_hier(rb, chmax)
        c = _get_i32(ra, r)
        active = _get_f32(cneg, c) == 0.0
        return r, c, gmax, active

    def w_cond(carry):
        return jnp.logical_not(carry[3])

    def w_body(carry):
        _refresh(src_hbm, carry[0], rb, ra, cneg, chmax, rowbuf, sem)
        return pop()

    def iter_body(i, _):
        r, c, v, _a = jax.lax.while_loop(w_cond, w_body, pop())
        record(i, r, c, v)
        _set_elem(rb, r, jnp.float32(NEG_INF))
        _update_chmax(rb, chmax, _chunk_of(r))
        _set_elem(cneg, c, jnp.float32(NEG_INF))
        return 0

    jax.lax.fori_loop(0, num_valid, iter_body, 0)


def _sc_match(sim, simT, rowmax, rowarg, colmax, colarg, colsum, refmask,
              osize, invnr):
    f32 = jnp.float32
    i32 = jnp.int32
    out_type = [
        jax.ShapeDtypeStruct((NPTS,), f32),
        jax.ShapeDtypeStruct((NPTS,), f32),
        jax.ShapeDtypeStruct((NPTS,), f32),
        jax.ShapeDtypeStruct((NPTS,), i32),
    ]
    scratch_types = [
        pltpu.VMEM((N,), f32),
        pltpu.VMEM((N,), i32),
        pltpu.VMEM((N,), f32),
        pltpu.VMEM((NCH,), f32),
        pltpu.VMEM((N,), f32),
        pltpu.VMEM((N,), i32),
        pltpu.VMEM((N,), f32),
        pltpu.VMEM((N,), f32),
        pltpu.VMEM((N,), i32),
        pltpu.VMEM((N,), i32),
        pltpu.VMEM((N,), i32),
        pltpu.VMEM((N,), f32),
        pltpu.VMEM((N,), i32),
        pltpu.VMEM((N,), f32),
        pltpu.VMEM((N,), i32),
        pltpu.VMEM((N,), f32),
        pltpu.VMEM((N,), i32),
        pltpu.VMEM((L,), i32),
        pltpu.VMEM((L,), f32),
        pltpu.VMEM((NPTS,), f32),
        pltpu.VMEM((NPTS,), f32),
        pltpu.VMEM((NPTS,), f32),
        pltpu.VMEM((NPTS,), i32),
        pltpu.SemaphoreType.DMA,
    ]

    @pl.kernel(out_type=out_type,
               mesh=plsc.VectorSubcoreMesh(core_axis_name="c",
                                           subcore_axis_name="s"),
               scratch_types=scratch_types,
               compiler_params=_SC_PARAMS)
    def run(sim_ref, simT_ref, rowmax_ref, rowarg_ref, colmax_ref, colarg_ref,
            colsum_ref, refmask_ref, osize_ref, invnr_ref, xo, yo, so, lo,
            rb, ra, cneg, chmax, rowbuf, colof, scoreof, slotscore, slottgt,
            fwslot, bwc, fgscore, fgtgt, colmax_s, colarg_s, colsum_s,
            refmask_s, osize_s, invnr_s, xs, ys, ss, ls, sem):
        cid = jax.lax.axis_index("c")
        sid = jax.lax.axis_index("s")

        @pl.when(jnp.logical_and(cid == 0, sid == 0))
        def _():
            lanes = _lanes()
            pltpu.async_copy(rowmax_ref, rb, sem).wait()
            pltpu.async_copy(rowarg_ref, ra, sem).wait()
            pltpu.async_copy(colmax_ref, colmax_s, sem).wait()
            pltpu.async_copy(colarg_ref, colarg_s, sem).wait()
            pltpu.async_copy(colsum_ref, colsum_s, sem).wait()
            pltpu.async_copy(refmask_ref, refmask_s, sem).wait()
            pltpu.async_copy(osize_ref, osize_s, sem).wait()
            pltpu.async_copy(invnr_ref, invnr_s, sem).wait()

            _fill(cneg, 0.0)
            _fill(slotscore, NEG_INF)
            _fill(slottgt, 0)
            _fill(fwslot, 0)
            _fill(bwc, 0)
            _fill(fgscore, NEG_INF)
            _fill(fgtgt, 0)
            _fill(xs, 0.0)
            _fill(ys, 0.0)
            _fill(ss, 0.0)
            _fill(ls, 0)

            def nr_body(q, acc):
                m = refmask_s[pl.ds(q * L, L)].astype(f32)
                _update_chmax(rb, chmax, q)
                return acc + jnp.sum(m)
            num_ref_f = jax.lax.fori_loop(0, NCH, nr_body, f32(0.0))
            num_ref = num_ref_f.astype(i32)

            def rec_f(i, r, c, v):
                _set_elem(colof, r, c)
                _set_elem(scoreof, r, v)
            _greedy(sim_ref, num_ref, rb, ra, cneg, chmax, rowbuf, sem, rec_f)

            def fc_body(r, cnt):
                is_ref = _get_i32(refmask_s, r) == 1

                @pl.when(is_ref)
                def _():
                    c = _get_i32(colof, r)
                    _set_elem(slotscore, cnt, _get_f32(scoreof, r))
                    _set_elem(slottgt, cnt, c)
                    _set_elem(fwslot, c, cnt)
                return cnt + jnp.where(is_ref, 1, 0).astype(i32)
            jax.lax.fori_loop(0, N, fc_body, i32(0))

            def bi_body(q, _):
                sl = pl.ds(q * L, L)
                picked = cneg[sl] < 0.0
                rb[sl] = jnp.where(picked, colmax_s[sl], f32(NEG_INF))
                ra[sl] = colarg_s[sl]
                cneg[sl] = jnp.zeros((L,), f32)
                _update_chmax(rb, chmax, q)
                return 0
            jax.lax.fori_loop(0, NCH, bi_body, 0)

            def rec_b(i, t, c, v):
                s = _get_i32(fwslot, t)
                _set_elem(bwc, s, c)
            _greedy(simT_ref, num_ref, rb, ra, cneg, chmax, rowbuf, sem,
                    rec_b)

            def vc_body(s, k):
                bc = _get_i32(bwc, s)
                ok = jnp.logical_and(s < num_ref,
                                     _get_i32(refmask_s, bc) == 1)

                @pl.when(ok)
                def _():
                    _set_elem(fgscore, k, _get_f32(slotscore, s))
                    _set_elem(fgtgt, k, _get_i32(slottgt, s))
                return k + jnp.where(ok, 1, 0).astype(i32)
            k = jax.lax.fori_loop(0, N, vc_body, i32(0))

            @pl.when(k == 0)
            def _():
                bv, b = _argmax_ref(slotscore)
                _set_elem(fgscore, 0, bv)
                _set_elem(fgtgt, 0, _get_i32(slottgt, b))

            n_fg = jnp.where(k == 0, i32(1), k)
            nf = jnp.minimum(i32(NUM_FG), n_fg)

            osv = osize_s[pl.ds(0, L)]
            h_f = jnp.max(jnp.where(lanes == 0, osv, i32(-BIG_I))).astype(f32)
            w_f = jnp.max(jnp.where(lanes == 1, osv, i32(-BIG_I))).astype(f32)

            def top_body(p, _):
                gmax, slot = _argmax_ref(fgscore)
                _set_elem(fgscore, slot, f32(NEG_INF))
                tgt = _get_i32(fgtgt, slot)
                ok = p < nf
                fx = ((jnp.bitwise_and(tgt, FEAT - 1).astype(f32) + 0.5)
                      * f32(1.0 / FEAT) * w_f)
                fy = ((jax.lax.shift_right_logical(tgt, 5).astype(f32) + 0.5)
                      * f32(1.0 / FEAT) * h_f)
                _set_elem(xs, p, jnp.where(ok, fx, f32(0.0)))
                _set_elem(ys, p, jnp.where(ok, fy, f32(0.0)))
                _set_elem(ss, p, jnp.where(ok, gmax, f32(0.0)))
                _set_elem(ls, p, jnp.where(ok, i32(1), i32(-1)))
                return 0
            jax.lax.fori_loop(0, NUM_FG, top_body, 0)

            inv_nr = jnp.max(invnr_s[pl.ds(0, L)])
            def avg_body(q, _):
                sl = pl.ds(q * L, L)
                rowbuf[sl] = -colsum_s[sl]
                return 0
            jax.lax.fori_loop(0, NCH, avg_body, 0)
            for q in range(NUM_BG):
                gmax, idx = _argmax_ref(rowbuf)
                _set_elem(rowbuf, idx, f32(NEG_INF))
                p = NUM_FG + q
                bx = ((jnp.bitwise_and(idx, FEAT - 1).astype(f32) + 0.5)
                      * f32(1.0 / FEAT) * w_f)
                by = ((jax.lax.shift_right_logical(idx, 5).astype(f32) + 0.5)
                      * f32(1.0 / FEAT) * h_f)
                _set_elem(xs, p, bx)
                _set_elem(ys, p, by)
                _set_elem(ss, p, -gmax * inv_nr)
                _set_elem(ls, p, i32(0))

            pltpu.async_copy(xs, xo, sem).wait()
            pltpu.async_copy(ys, yo, sem).wait()
            pltpu.async_copy(ss, so, sem).wait()
            pltpu.async_copy(ls, lo, sem).wait()

    return run(sim, simT, rowmax, rowarg, colmax, colarg, colsum, refmask,
               osize, invnr)


def kernel(similarity_map, ref_mask, original_size):
    f32 = jnp.float32
    i32 = jnp.int32
    sim = similarity_map.astype(f32)
    refneg = jnp.where(ref_mask, f32(0.0), f32(NEG_INF)).reshape(N, 1)
    rowmax, rowarg, colmax, colarg, colsum, inv = pl.pallas_call(
        _prep_body,
        out_shape=[
            jax.ShapeDtypeStruct((N, 1), f32),
            jax.ShapeDtypeStruct((N, 1), i32),
            jax.ShapeDtypeStruct((1, N), f32),
            jax.ShapeDtypeStruct((1, N), i32),
            jax.ShapeDtypeStruct((1, N), f32),
            jax.ShapeDtypeStruct((1, 128), f32),
        ],
    )(sim, refneg)
    simT = sim.T
    refmask_i32 = ref_mask.astype(i32)
    osize = jnp.zeros((L,), i32)
    osize = osize.at[0].set(original_size[0].astype(i32))
    osize = osize.at[1].set(original_size[1].astype(i32))
    x, y, s, labels = _sc_match(
        sim, simT, rowmax.reshape(N), rowarg.reshape(N), colmax.reshape(N),
        colarg.reshape(N), colsum.reshape(N), refmask_i32, osize,
        inv.reshape(128)[:L])
    m = NUM_FG + NUM_BG
    points = jnp.stack([x[:m], y[:m], s[:m]], axis=1)
    return points, labels[:m]
